# baseline (speedup 1.0000x reference)
import jax
import jax.numpy as jnp
from jax import lax
from jax.experimental import pallas as pl
from jax.experimental.pallas import tpu as pltpu


def _route_kernel(eit_ref, ewt_ref, call_ref, used_ref, nu_ref):
    ne, t = call_ref.shape
    k = eit_ref.shape[0]
    io_e = lax.broadcasted_iota(jnp.int32, (ne, t), 0)
    call = jnp.zeros((ne, t), jnp.float32)
    routed = jnp.zeros((ne, t), jnp.float32)
    for kk in range(k):
        oh = (io_e == jnp.broadcast_to(eit_ref[kk:kk + 1, :], (ne, t)))
        ohf = oh.astype(jnp.float32)
        call += ohf * jnp.broadcast_to(ewt_ref[kk:kk + 1, :], (ne, t))
        routed += ohf
    call_ref[...] = call
    m_col = (jnp.sum(routed, axis=1, keepdims=True) > 0.0)
    m_f = m_col.astype(jnp.float32)
    tri = (lax.broadcasted_iota(jnp.int32, (ne, ne), 0)
           >= lax.broadcasted_iota(jnp.int32, (ne, ne), 1)
           ).astype(jnp.float32)
    pos_f = jnp.dot(tri, m_f, preferred_element_type=jnp.float32) - 1.0
    pos = pos_f.astype(jnp.int32)
    io_slot = lax.broadcasted_iota(jnp.int32, (ne, ne), 1)
    sel = jnp.logical_and(jnp.broadcast_to(pos, (ne, ne)) == io_slot,
                          jnp.broadcast_to(m_col, (ne, ne)))
    eids = lax.broadcasted_iota(jnp.int32, (ne, ne), 0).astype(jnp.float32)
    used_f = jnp.sum(sel.astype(jnp.float32) * eids, axis=0, keepdims=True)
    used_ref[...] = used_f.astype(jnp.int32)
    nu_ref[...] = jnp.sum(m_f, axis=0, keepdims=True).astype(jnp.int32)


def _ffn_kernel(used_ref, nu_ref, x_ref, call_ref, w1_ref, w2_ref, w3_ref,
                out_ref):
    i = pl.program_id(0)

    @pl.when(i == 0)
    def _init():
        out_ref[...] = jnp.zeros_like(out_ref)

    @pl.when(i < nu_ref[0, 0])
    def _body():
        x = x_ref[...]
        g = jnp.dot(x, w1_ref[0], preferred_element_type=jnp.float32)
        u = jnp.dot(x, w3_ref[0], preferred_element_type=jnp.float32)
        h = g * jax.lax.logistic(g) * u
        y = jnp.dot(h, w2_ref[0], preferred_element_type=jnp.float32)
        c = call_ref[used_ref[0, i], :]
        out_ref[...] += c[:, None] * y


def _expert_block(i, used, nu):
    return used[0, jnp.minimum(i, nu[0, 0] - 1)]


@jax.jit
def kernel(x, expert_indices, expert_weights, w1_stacked, w2_stacked,
           w3_stacked):
    t, h = x.shape
    e, _, inter = w1_stacked.shape
    k = expert_indices.shape[1]
    n = t * k

    eit = expert_indices.astype(jnp.int32).T
    ewt = expert_weights.T
    call, used, nu = pl.pallas_call(
        _route_kernel,
        out_shape=[
            jax.ShapeDtypeStruct((e, t), jnp.float32),
            jax.ShapeDtypeStruct((1, e), jnp.int32),
            jax.ShapeDtypeStruct((1, 1), jnp.int32),
        ],
    )(eit, ewt)

    grid_spec = pltpu.PrefetchScalarGridSpec(
        num_scalar_prefetch=2,
        grid=(n,),
        in_specs=[
            pl.BlockSpec((t, h), lambda i, used, nu: (0, 0)),
            pl.BlockSpec((e, t), lambda i, used, nu: (0, 0)),
            pl.BlockSpec((1, h, inter),
                         lambda i, used, nu: (_expert_block(i, used, nu), 0, 0)),
            pl.BlockSpec((1, inter, h),
                         lambda i, used, nu: (_expert_block(i, used, nu), 0, 0)),
            pl.BlockSpec((1, h, inter),
                         lambda i, used, nu: (_expert_block(i, used, nu), 0, 0)),
        ],
        out_specs=pl.BlockSpec((t, h), lambda i, used, nu: (0, 0)),
    )
    return pl.pallas_call(
        _ffn_kernel,
        grid_spec=grid_spec,
        out_shape=jax.ShapeDtypeStruct((t, h), jnp.float32),
    )(used, nu, x, call, w1_stacked, w2_stacked, w3_stacked)
